# baseline (speedup 1.0000x reference)
import functools

import jax
import jax.numpy as jnp
from jax import lax
from jax.experimental import pallas as pl
from jax.experimental.pallas import tpu as pltpu
from jax.experimental.pallas import tpu_sc as plsc

_B, _C, _H, _W, _N = 2, 96, 224, 224, 4
_HW = _H * _W
_TP = 512
_NW = 32
_PPW = _B * _HW // _NW
_SCT = 64
_SCNT = _PPW // _SCT
_CL = _C // 16


def _tc_proj(x_ref, wq_ref, bq_ref, woff_ref, boff_ref, wkv_ref, bkv_ref,
             q_ref, kv_ref, idx_ref):
    b = pl.program_id(0)
    pb = pl.program_id(1)
    xt = x_ref[0]
    q = lax.dot_general(xt, wq_ref[...], (((0,), (1,)), ((), ())),
                        preferred_element_type=jnp.float32) + bq_ref[...]
    q_ref[0] = q
    offs = jnp.transpose(
        lax.dot_general(q, woff_ref[...], (((1,), (1,)), ((), ())),
                        preferred_element_type=jnp.float32) + boff_ref[...])
    kv = lax.dot_general(xt, wkv_ref[...], (((0,), (0,)), ((), ())),
                         preferred_element_type=jnp.float32) + bkv_ref[...]
    kv_ref[0] = kv.astype(jnp.bfloat16)
    p = pb * _TP + lax.broadcasted_iota(jnp.int32, (1, _TP), 1)
    px = (p % _W).astype(jnp.float32)
    py = (p // _W).astype(jnp.float32)
    base = b * _HW
    rows = []
    for n in range(_N):
        rx = jnp.clip(jnp.round(px + offs[2 * n:2 * n + 1, :]).astype(jnp.int32),
                      0, _W - 1)
        ry = jnp.clip(jnp.round(py + offs[2 * n + 1:2 * n + 2, :]).astype(jnp.int32),
                      0, _H - 1)
        rows.append(base + ry * _W + rx)
    idx_cat = jnp.concatenate(rows, axis=0)
    for st in range(_TP // _SCT):
        idx_ref[0, st] = idx_cat[:, _SCT * st:_SCT * (st + 1)]


def _tc_stage(x2, Wq, bq2, Woff, boff2, Wkvt, bkv2, interpret=False):
    return pl.pallas_call(
        _tc_proj,
        grid=(_B, _HW // _TP),
        in_specs=[
            pl.BlockSpec((1, _C, _TP), lambda b, p: (b, 0, p)),
            pl.BlockSpec((_C, _C), lambda b, p: (0, 0)),
            pl.BlockSpec((1, _C), lambda b, p: (0, 0)),
            pl.BlockSpec((2 * _N, _C), lambda b, p: (0, 0)),
            pl.BlockSpec((1, 2 * _N), lambda b, p: (0, 0)),
            pl.BlockSpec((_C, 2 * _C), lambda b, p: (0, 0)),
            pl.BlockSpec((1, 2 * _C), lambda b, p: (0, 0)),
        ],
        out_specs=[
            pl.BlockSpec((1, _TP, _C), lambda b, p: (b, p, 0)),
            pl.BlockSpec((1, _TP, 2 * _C), lambda b, p: (b, p, 0)),
            pl.BlockSpec((1, _TP // _SCT, _N, _SCT), lambda b, p: (b, p, 0, 0)),
        ],
        out_shape=[
            jax.ShapeDtypeStruct((_B, _HW, _C), jnp.float32),
            jax.ShapeDtypeStruct((_B, _HW, 2 * _C), jnp.bfloat16),
            jax.ShapeDtypeStruct((_B, _HW // _SCT, _N, _SCT), jnp.int32),
        ],
        interpret=interpret,
    )(x2, Wq, bq2, Woff, boff2, Wkvt, bkv2)


def _sc_attn_body(q_hbm, kv_hbm, idx_hbm, out_hbm,
                  idx_v, kv_v, q_v, o_v, sem0, sem1, isem, osem0, osem1):
    cid = lax.axis_index("c")
    sid = lax.axis_index("s")
    wid = sid * 2 + cid
    g0 = wid * _PPW
    t0g = wid * _SCNT
    sems = (sem0, sem1)
    osems = (osem0, osem1)

    def _idx_copy(t, sl):
        return pltpu.make_async_copy(idx_hbm.at[t0g + t], idx_v.at[sl], isem)

    def _gather_copies(t, ph, sl):
        gp = g0 + t * _SCT
        cps = [
            pltpu.make_async_copy(kv_hbm.at[idx_v.at[sl, n]],
                                  kv_v.at[ph, pl.ds(n * _SCT, _SCT)],
                                  sems[ph])
            for n in range(_N)
        ]
        cps.append(pltpu.make_async_copy(q_hbm.at[pl.ds(gp, _SCT)],
                                         q_v.at[ph], sems[ph]))
        return cps

    def _ostore(t, ph):
        return pltpu.make_async_copy(o_v.at[ph],
                                     out_hbm.at[pl.ds(g0 + t * _SCT, _SCT)],
                                     osems[ph])

    perms = [lax.iota(jnp.int32, 16) ^ st for st in (8, 4, 2, 1)]

    def compute(t, ph):
        if isinstance(t, int):
            if t >= 2:
                _ostore(t - 2, ph).wait()
        else:
            @pl.when(t >= 2)
            def _():
                _ostore(t - 2, ph).wait()

        def pixel_body(p, carry):
            qr = [q_v[ph, p, pl.ds(16 * k, 16)] for k in range(_CL)]
            s = []
            for n in range(_N):
                row = n * _SCT + p
                acc = None
                for k in range(_CL // 2):
                    kraw = kv_v[ph, row, pl.ds(32 * k, 32)]
                    ka, kb = plsc.unpack(kraw,
                                         format=plsc.PackFormat.INTERLEAVED)
                    term = qr[2 * k] * ka + qr[2 * k + 1] * kb
                    acc = term if acc is None else acc + term
                for pm in perms:
                    acc = acc + jnp.take_along_axis(acc, pm, axis=0)
                s.append(acc)
            m = jnp.maximum(jnp.maximum(s[0], s[1]), jnp.maximum(s[2], s[3]))
            e = [jnp.exp(si - m) for si in s]
            d = e[0] + e[1] + e[2] + e[3]
            w = [ei / d for ei in e]
            oacc = [None] * _CL
            for n in range(_N):
                row = n * _SCT + p
                for k in range(_CL // 2):
                    vraw = kv_v[ph, row, pl.ds(_C + 32 * k, 32)]
                    va, vb = plsc.unpack(vraw,
                                         format=plsc.PackFormat.INTERLEAVED)
                    ta = w[n] * va
                    tb = w[n] * vb
                    oacc[2 * k] = ta if oacc[2 * k] is None else oacc[2 * k] + ta
                    oacc[2 * k + 1] = (tb if oacc[2 * k + 1] is None
                                       else oacc[2 * k + 1] + tb)
            for k in range(_CL):
                o_v[ph, p, pl.ds(16 * k, 16)] = oacc[k]
            return carry

        lax.fori_loop(0, _SCT, pixel_body, 0)
        _ostore(t, ph).start()

    _idx_copy(0, 0).start()
    _idx_copy(0, 0).wait()
    for cp in _gather_copies(0, 0, 0):
        cp.start()
    _idx_copy(1, 1).start()

    def body4(j, carry):
        for r in range(4):
            t = 4 * j + r
            _idx_copy(t + 1, (r + 1) % 4).wait()
            for cp in _gather_copies(t + 1, (r + 1) % 2, (r + 1) % 4):
                cp.start()

            @pl.when(t + 2 < _SCNT)
            def _():
                _idx_copy(t + 2, (r + 2) % 4).start()

            for cp in _gather_copies(t, r % 2, r % 4):
                cp.wait()
            compute(t, r % 2)
        return carry

    lax.fori_loop(0, (_SCNT - 1) // 4, body4, 0)
    tl = _SCNT - 1
    for cp in _gather_copies(tl, tl % 2, tl % 4):
        cp.wait()
    compute(tl, tl % 2)
    _ostore(tl - 1, (tl - 1) % 2).wait()
    _ostore(tl, tl % 2).wait()


@functools.cache
def _sc_attn():
    return pl.kernel(
        _sc_attn_body,
        out_type=jax.ShapeDtypeStruct((_B * _HW, _C), jnp.float32),
        mesh=plsc.VectorSubcoreMesh(core_axis_name="c", subcore_axis_name="s"),
        compiler_params=pltpu.CompilerParams(use_tc_tiling_on_sc=False,
                                             needs_layout_passes=False),
        scratch_types=[
            pltpu.VMEM((4, _N, _SCT), jnp.int32),
            pltpu.VMEM((2, _N * _SCT, 2 * _C), jnp.bfloat16),
            pltpu.VMEM((2, _SCT, _C), jnp.float32),
            pltpu.VMEM((2, _SCT, _C), jnp.float32),
            pltpu.SemaphoreType.DMA,
            pltpu.SemaphoreType.DMA,
            pltpu.SemaphoreType.DMA,
            pltpu.SemaphoreType.DMA,
            pltpu.SemaphoreType.DMA,
        ],
    )


_PKV = [32 * kb + (i // 2 if i % 2 == 0 else 16 + i // 2)
        for kb in range(2 * _C // 32) for i in range(32)]


def kernel(x, Wq, bq, Wk, bk, Wv, bv, Woff, boff):
    x2 = x.reshape(_B, _C, _HW)
    pkv = jnp.asarray(_PKV)
    Wkvt = jnp.concatenate([Wk.T, Wv.T], axis=1)[:, pkv]
    bkv2 = jnp.concatenate([bk, bv]).reshape(1, 2 * _C)[:, pkv]
    bq2 = bq.reshape(1, _C)
    boff2 = boff.reshape(1, 2 * _N)
    q, kv, idx = _tc_stage(x2, Wq, bq2, Woff, boff2, Wkvt, bkv2)
    out = _sc_attn()(q.reshape(_B * _HW, _C),
                     kv.reshape(_B * _HW, 2 * _C),
                     idx.reshape(_B * _HW // _SCT, _N, _SCT))
    return out.reshape(_B, _HW, _C).swapaxes(1, 2).reshape(_B, _C, _H, _W)


# speedup vs baseline: 1.6172x; 1.6172x over previous
import functools

import jax
import jax.numpy as jnp
from jax import lax
from jax.experimental import pallas as pl
from jax.experimental.pallas import tpu as pltpu
from jax.experimental.pallas import tpu_sc as plsc

_B, _C, _H, _W, _N = 2, 96, 224, 224, 4
_HW = _H * _W
_TP = 512
_NW = 32
_PPW = _B * _HW // _NW
_SCT = 64
_SCNT = _PPW // _SCT
_CL = _C // 16


def _tc_proj(x_ref, wq_ref, bq_ref, woff_ref, boff_ref, wlo_ref, blo_ref,
             whi_ref, bhi_ref, q_ref, kv_ref, idx_ref):
    b = pl.program_id(0)
    pb = pl.program_id(1)
    xt = x_ref[0]
    q = lax.dot_general(xt, wq_ref[...], (((0,), (1,)), ((), ())),
                        preferred_element_type=jnp.float32) + bq_ref[...]
    q_ref[0] = q
    offs = jnp.transpose(
        lax.dot_general(q, woff_ref[...], (((1,), (1,)), ((), ())),
                        preferred_element_type=jnp.float32) + boff_ref[...])
    kvlo = lax.dot_general(xt, wlo_ref[...], (((0,), (0,)), ((), ())),
                           preferred_element_type=jnp.float32) + blo_ref[...]
    kvhi = lax.dot_general(xt, whi_ref[...], (((0,), (0,)), ((), ())),
                           preferred_element_type=jnp.float32) + bhi_ref[...]
    lo = lax.bitcast_convert_type(kvlo.astype(jnp.bfloat16),
                                  jnp.int16).astype(jnp.int32) & 0xFFFF
    hi = lax.bitcast_convert_type(kvhi.astype(jnp.bfloat16),
                                  jnp.int16).astype(jnp.int32)
    kv_ref[0] = (hi << 16) | lo
    p = pb * _TP + lax.broadcasted_iota(jnp.int32, (1, _TP), 1)
    px = (p % _W).astype(jnp.float32)
    py = (p // _W).astype(jnp.float32)
    base = b * _HW
    rows = []
    for n in range(_N):
        rx = jnp.clip(jnp.round(px + offs[2 * n:2 * n + 1, :]).astype(jnp.int32),
                      0, _W - 1)
        ry = jnp.clip(jnp.round(py + offs[2 * n + 1:2 * n + 2, :]).astype(jnp.int32),
                      0, _H - 1)
        rows.append(base + ry * _W + rx)
    idx_cat = jnp.concatenate(rows, axis=0)
    for st in range(_TP // _SCT):
        idx_ref[0, st] = idx_cat[:, _SCT * st:_SCT * (st + 1)]


def _tc_stage(x2, Wq, bq2, Woff, boff2, Wlo, blo, Whi, bhi, interpret=False):
    return pl.pallas_call(
        _tc_proj,
        grid=(_B, _HW // _TP),
        in_specs=[
            pl.BlockSpec((1, _C, _TP), lambda b, p: (b, 0, p)),
            pl.BlockSpec((_C, _C), lambda b, p: (0, 0)),
            pl.BlockSpec((1, _C), lambda b, p: (0, 0)),
            pl.BlockSpec((2 * _N, _C), lambda b, p: (0, 0)),
            pl.BlockSpec((1, 2 * _N), lambda b, p: (0, 0)),
            pl.BlockSpec((_C, 128), lambda b, p: (0, 0)),
            pl.BlockSpec((1, 128), lambda b, p: (0, 0)),
            pl.BlockSpec((_C, 128), lambda b, p: (0, 0)),
            pl.BlockSpec((1, 128), lambda b, p: (0, 0)),
        ],
        out_specs=[
            pl.BlockSpec((1, _TP, _C), lambda b, p: (b, p, 0)),
            pl.BlockSpec((1, _TP, 128), lambda b, p: (b, p, 0)),
            pl.BlockSpec((1, _TP // _SCT, _N, _SCT), lambda b, p: (b, p, 0, 0)),
        ],
        out_shape=[
            jax.ShapeDtypeStruct((_B, _HW, _C), jnp.float32),
            jax.ShapeDtypeStruct((_B, _HW, 128), jnp.int32),
            jax.ShapeDtypeStruct((_B, _HW // _SCT, _N, _SCT), jnp.int32),
        ],
        interpret=interpret,
    )(x2, Wq, bq2, Woff, boff2, Wlo, blo, Whi, bhi)


def _sc_attn_body(q_hbm, kv_hbm, idx_hbm, out_hbm,
                  idx_v, kv_v, q_v, o_v, sem0, sem1, isem, osem0, osem1):
    cid = lax.axis_index("c")
    sid = lax.axis_index("s")
    wid = sid * 2 + cid
    g0 = wid * _PPW
    t0g = wid * _SCNT
    sems = (sem0, sem1)
    osems = (osem0, osem1)

    def _idx_copy(t, sl):
        return pltpu.make_async_copy(idx_hbm.at[t0g + t], idx_v.at[sl], isem)

    def _gather_copies(t, ph, sl):
        gp = g0 + t * _SCT
        cps = [
            pltpu.make_async_copy(kv_hbm.at[idx_v.at[sl, n]],
                                  kv_v.at[ph, pl.ds(n * _SCT, _SCT)],
                                  sems[ph])
            for n in range(_N)
        ]
        cps.append(pltpu.make_async_copy(q_hbm.at[pl.ds(gp, _SCT)],
                                         q_v.at[ph], sems[ph]))
        return cps

    def _ostore(t, ph):
        return pltpu.make_async_copy(o_v.at[ph],
                                     out_hbm.at[pl.ds(g0 + t * _SCT, _SCT)],
                                     osems[ph])

    perms = [lax.iota(jnp.int32, 16) ^ st for st in (8, 4, 2, 1)]

    def compute(t, ph):
        if isinstance(t, int):
            if t >= 2:
                _ostore(t - 2, ph).wait()
        else:
            @pl.when(t >= 2)
            def _():
                _ostore(t - 2, ph).wait()

        def pixel_body(p, carry):
            qr = [q_v[ph, p, pl.ds(16 * k, 16)] for k in range(_CL)]
            s = []
            for n in range(_N):
                row = n * _SCT + p
                acc = None
                for k in range(_CL // 2):
                    kraw = plsc.bitcast(kv_v[ph, row, pl.ds(16 * k, 16)],
                                        jnp.bfloat16)
                    ka, kb = plsc.unpack(kraw,
                                         format=plsc.PackFormat.INTERLEAVED)
                    term = qr[2 * k] * ka + qr[2 * k + 1] * kb
                    acc = term if acc is None else acc + term
                for pm in perms:
                    acc = acc + jnp.take_along_axis(acc, pm, axis=0)
                s.append(acc)
            m = jnp.maximum(jnp.maximum(s[0], s[1]), jnp.maximum(s[2], s[3]))
            e = [jnp.exp(si - m) for si in s]
            d = e[0] + e[1] + e[2] + e[3]
            w = [ei / d for ei in e]
            oacc = [None] * _CL
            for n in range(_N):
                row = n * _SCT + p
                for k in range(_CL // 2):
                    vraw = plsc.bitcast(
                        kv_v[ph, row, pl.ds(_C // 2 + 16 * k, 16)],
                        jnp.bfloat16)
                    va, vb = plsc.unpack(vraw,
                                         format=plsc.PackFormat.INTERLEAVED)
                    ta = w[n] * va
                    tb = w[n] * vb
                    oacc[2 * k] = ta if oacc[2 * k] is None else oacc[2 * k] + ta
                    oacc[2 * k + 1] = (tb if oacc[2 * k + 1] is None
                                       else oacc[2 * k + 1] + tb)
            for k in range(_CL):
                o_v[ph, p, pl.ds(16 * k, 16)] = oacc[k]
            return carry

        lax.fori_loop(0, _SCT, pixel_body, 0)
        _ostore(t, ph).start()

    _idx_copy(0, 0).start()
    _idx_copy(0, 0).wait()
    for cp in _gather_copies(0, 0, 0):
        cp.start()
    _idx_copy(1, 1).start()

    def body4(j, carry):
        for r in range(4):
            t = 4 * j + r
            _idx_copy(t + 1, (r + 1) % 4).wait()
            for cp in _gather_copies(t + 1, (r + 1) % 2, (r + 1) % 4):
                cp.start()

            @pl.when(t + 2 < _SCNT)
            def _():
                _idx_copy(t + 2, (r + 2) % 4).start()

            for cp in _gather_copies(t, r % 2, r % 4):
                cp.wait()
            compute(t, r % 2)
        return carry

    lax.fori_loop(0, (_SCNT - 1) // 4, body4, 0)
    tl = _SCNT - 1
    for cp in _gather_copies(tl, tl % 2, tl % 4):
        cp.wait()
    compute(tl, tl % 2)
    _ostore(tl - 1, (tl - 1) % 2).wait()
    _ostore(tl, tl % 2).wait()


@functools.cache
def _sc_attn():
    return pl.kernel(
        _sc_attn_body,
        out_type=jax.ShapeDtypeStruct((_B * _HW, _C), jnp.float32),
        mesh=plsc.VectorSubcoreMesh(core_axis_name="c", subcore_axis_name="s"),
        compiler_params=pltpu.CompilerParams(needs_layout_passes=False),
        scratch_types=[
            pltpu.VMEM((4, _N, _SCT), jnp.int32),
            pltpu.VMEM((2, _N * _SCT, 128), jnp.int32),
            pltpu.VMEM((2, _SCT, _C), jnp.float32),
            pltpu.VMEM((2, _SCT, _C), jnp.float32),
            pltpu.SemaphoreType.DMA,
            pltpu.SemaphoreType.DMA,
            pltpu.SemaphoreType.DMA,
            pltpu.SemaphoreType.DMA,
            pltpu.SemaphoreType.DMA,
        ],
    )


_PKV = [32 * kb + (i // 2 if i % 2 == 0 else 16 + i // 2)
        for kb in range(2 * _C // 32) for i in range(32)]


def kernel(x, Wq, bq, Wk, bk, Wv, bv, Woff, boff):
    x2 = x.reshape(_B, _C, _HW)
    pkv = jnp.asarray(_PKV)
    Wkvt = jnp.concatenate([Wk.T, Wv.T], axis=1)[:, pkv]
    bkvt = jnp.concatenate([bk, bv]).reshape(1, 2 * _C)[:, pkv]
    Wpad = jnp.pad(Wkvt, ((0, 0), (0, 64)))
    bpad = jnp.pad(bkvt, ((0, 0), (0, 64)))
    Wlo, Whi = Wpad[:, 0::2], Wpad[:, 1::2]
    blo, bhi = bpad[:, 0::2], bpad[:, 1::2]
    bq2 = bq.reshape(1, _C)
    boff2 = boff.reshape(1, 2 * _N)
    q, kv, idx = _tc_stage(x2, Wq, bq2, Woff, boff2, Wlo, blo, Whi, bhi)
    out = _sc_attn()(q.reshape(_B * _HW, _C),
                     kv.reshape(_B * _HW, 128),
                     idx.reshape(_B * _HW // _SCT, _N, _SCT))
    return out.reshape(_B, _HW, _C).swapaxes(1, 2).reshape(_B, _C, _H, _W)
